# Initial kernel scaffold; baseline (speedup 1.0000x reference)
import functools

import jax
import jax.numpy as jnp
from jax import lax
from jax.experimental import pallas as pl
from jax.experimental.pallas import tpu as pltpu
from jax.experimental.pallas import tpu_sc as plsc

_B = 16384
_H = 50
_D = 64
_N = _B * _H
_CH = 128


@functools.cache
def _build(n_per_w, n_chunks, nc):
    mesh = plsc.VectorSubcoreMesh(core_axis_name="c", subcore_axis_name="s")

    @functools.partial(
        pl.kernel,
        mesh=mesh,
        out_type=jax.ShapeDtypeStruct((_N, _D), jnp.float32),
        scratch_types=[
            pltpu.VMEM((n_chunks, _CH), jnp.int32),
            pltpu.VMEM((_CH, _D), jnp.float32),
            pltpu.SemaphoreType.DMA,
        ],
    )
    def gather_kernel(idx_hbm, table_hbm, out_hbm, idx_v, rows_v, sem):
        wid = lax.axis_index("s") * nc + lax.axis_index("c")
        base = wid * n_per_w
        pltpu.sync_copy(idx_hbm.at[pl.ds(wid * n_chunks, n_chunks)], idx_v)

        def body(c, carry):
            pltpu.async_copy(table_hbm.at[idx_v.at[c]], rows_v, sem).wait()
            pltpu.sync_copy(rows_v, out_hbm.at[pl.ds(base + c * _CH, _CH)])
            return carry

        lax.fori_loop(0, n_chunks, body, 0)

    return gather_kernel


def kernel(input_, weight):
    info = plsc.get_sparse_core_info()
    nw = info.num_cores * info.num_subcores
    n_per_w = _N // nw
    n_chunks = n_per_w // _CH
    idx2d = input_.reshape(_N // _CH, _CH).astype(jnp.int32)
    out = _build(n_per_w, n_chunks, info.num_cores)(idx2d, weight)
    return out.reshape(_B, _H, _D)


# speedup vs baseline: 1.6848x; 1.6848x over previous
import functools

import jax
import jax.numpy as jnp
from jax import lax
from jax.experimental import pallas as pl
from jax.experimental.pallas import tpu as pltpu
from jax.experimental.pallas import tpu_sc as plsc

_B = 16384
_H = 50
_D = 64
_N = _B * _H
_CH = 128


@functools.cache
def _build(n_per_w, n_chunks, nc):
    mesh = plsc.VectorSubcoreMesh(core_axis_name="c", subcore_axis_name="s")

    @functools.partial(
        pl.kernel,
        mesh=mesh,
        out_type=jax.ShapeDtypeStruct((_N, _D), jnp.float32),
        compiler_params=pltpu.CompilerParams(use_tc_tiling_on_sc=False),
        scratch_types=[
            pltpu.VMEM((n_chunks, _CH), jnp.int32),
            pltpu.VMEM((_CH, _D), jnp.float32),
            pltpu.SemaphoreType.DMA,
        ],
    )
    def gather_kernel(idx_hbm, table_hbm, out_hbm, idx_v, rows_v, sem):
        wid = lax.axis_index("s") * nc + lax.axis_index("c")
        base = wid * n_per_w
        pltpu.sync_copy(idx_hbm.at[pl.ds(wid * n_chunks, n_chunks)], idx_v)

        def body(c, carry):
            pltpu.async_copy(table_hbm.at[idx_v.at[c]], rows_v, sem).wait()
            pltpu.sync_copy(rows_v, out_hbm.at[pl.ds(base + c * _CH, _CH)])
            return carry

        lax.fori_loop(0, n_chunks, body, 0)

    return gather_kernel


def kernel(input_, weight):
    info = plsc.get_sparse_core_info()
    nw = info.num_cores * info.num_subcores
    n_per_w = _N // nw
    n_chunks = n_per_w // _CH
    idx2d = input_.reshape(_N // _CH, _CH).astype(jnp.int32)
    out = _build(n_per_w, n_chunks, info.num_cores)(idx2d, weight)
    return out.reshape(_B, _H, _D)


# speedup vs baseline: 1.8721x; 1.1112x over previous
import functools

import jax
import jax.numpy as jnp
from jax import lax
from jax.experimental import pallas as pl
from jax.experimental.pallas import tpu as pltpu
from jax.experimental.pallas import tpu_sc as plsc

_B = 16384
_H = 50
_D = 64
_G = 8


@functools.cache
def _build(rows_per_w, nc):
    mesh = plsc.VectorSubcoreMesh(core_axis_name="c", subcore_axis_name="s")
    n_groups = rows_per_w // _G
    n2 = n_groups // 2

    @functools.partial(
        pl.kernel,
        mesh=mesh,
        out_type=jax.ShapeDtypeStruct((_B, _H, _D), jnp.float32),
        compiler_params=pltpu.CompilerParams(use_tc_tiling_on_sc=False),
        scratch_types=[
            pltpu.VMEM((rows_per_w, _H), jnp.int32),
            pltpu.VMEM((_G, _H, _D), jnp.float32),
            pltpu.VMEM((_G, _H, _D), jnp.float32),
            pltpu.SemaphoreType.DMA,
            pltpu.SemaphoreType.DMA,
        ],
    )
    def gather_kernel(idx_hbm, table_hbm, out_hbm, idx_v, buf_a, buf_b, sem_a, sem_b):
        wid = lax.axis_index("s") * nc + lax.axis_index("c")
        row0 = wid * rows_per_w
        pltpu.sync_copy(idx_hbm.at[pl.ds(row0, rows_per_w)], idx_v)

        def gather(local_row, buf, sem):
            return pltpu.make_async_copy(
                table_hbm.at[idx_v.at[local_row]], buf, sem
            )

        for j in range(_G):
            gather(j, buf_a.at[j], sem_a).start()

        def body(h, carry):
            g0 = 2 * h
            for j in range(_G):
                gather((g0 + 1) * _G + j, buf_b.at[j], sem_b).start()
            for j in range(_G):
                gather(g0 * _G + j, buf_a.at[j], sem_a).wait()
                pltpu.sync_copy(buf_a.at[j], out_hbm.at[row0 + g0 * _G + j])
            @pl.when(h + 1 < n2)
            def _():
                for j in range(_G):
                    gather((g0 + 2) * _G + j, buf_a.at[j], sem_a).start()
            for j in range(_G):
                gather((g0 + 1) * _G + j, buf_b.at[j], sem_b).wait()
                pltpu.sync_copy(buf_b.at[j], out_hbm.at[row0 + (g0 + 1) * _G + j])
            return carry

        lax.fori_loop(0, n2, body, 0)

    return gather_kernel


def kernel(input_, weight):
    info = plsc.get_sparse_core_info()
    nw = info.num_cores * info.num_subcores
    rows_per_w = _B // nw
    return _build(rows_per_w, info.num_cores)(input_, weight)


# speedup vs baseline: 1.9705x; 1.0526x over previous
import functools

import jax
import jax.numpy as jnp
from jax import lax
from jax.experimental import pallas as pl
from jax.experimental.pallas import tpu as pltpu
from jax.experimental.pallas import tpu_sc as plsc

_B = 16384
_H = 50
_D = 64
_G = 8


@functools.cache
def _build(rows_per_w, nc):
    mesh = plsc.VectorSubcoreMesh(core_axis_name="c", subcore_axis_name="s")
    n_groups = rows_per_w // _G
    n2 = n_groups // 2

    @functools.partial(
        pl.kernel,
        mesh=mesh,
        out_type=jax.ShapeDtypeStruct((_B, _H, _D), jnp.float32),
        compiler_params=pltpu.CompilerParams(use_tc_tiling_on_sc=False),
        scratch_types=[
            pltpu.VMEM((rows_per_w, _H), jnp.int32),
            pltpu.VMEM((_G, _H, _D), jnp.float32),
            pltpu.VMEM((_G, _H, _D), jnp.float32),
            pltpu.SemaphoreType.DMA,
            pltpu.SemaphoreType.DMA,
        ],
    )
    def gather_kernel(idx_hbm, table_hbm, out_hbm, idx_v, buf_a, buf_b, sem_a, sem_b):
        wid = lax.axis_index("s") * nc + lax.axis_index("c")
        row0 = wid * rows_per_w
        pltpu.sync_copy(idx_hbm.at[pl.ds(row0, rows_per_w)], idx_v)

        def gather(local_row, buf, sem):
            return pltpu.make_async_copy(
                table_hbm.at[idx_v.at[local_row]], buf, sem
            )

        for j in range(_G):
            gather(j, buf_a.at[j], sem_a).start()

        def body(h, carry):
            g0 = 2 * h
            for j in range(_G):
                gather((g0 + 1) * _G + j, buf_b.at[j], sem_b).start()
            for j in range(_G):
                gather(g0 * _G + j, buf_a.at[j], sem_a).wait()
                pltpu.sync_copy(buf_a.at[j], out_hbm.at[row0 + g0 * _G + j])
            @pl.when(h + 1 < n2)
            def _():
                for j in range(_G):
                    gather((g0 + 2) * _G + j, buf_a.at[j], sem_a).start()
            for j in range(_G):
                gather((g0 + 1) * _G + j, buf_b.at[j], sem_b).wait()
                pltpu.sync_copy(buf_b.at[j], out_hbm.at[row0 + (g0 + 1) * _G + j])
            return carry

        lax.fori_loop(0, n2, body, 0)

    return gather_kernel


def kernel(input_, weight):
    info = plsc.get_sparse_core_info()
    nw = info.num_cores * info.num_subcores
    rows_per_w = _B // nw
    wpad = jnp.pad(weight, ((0, 0), (0, _D))).reshape(2 * 1000000, _D)
    idx2 = input_ * 2
    return _build(rows_per_w, info.num_cores)(idx2, wpad)


# speedup vs baseline: 2.7055x; 1.3730x over previous
import functools

import jax
import jax.numpy as jnp
from jax import lax
from jax.experimental import pallas as pl
from jax.experimental.pallas import tpu as pltpu
from jax.experimental.pallas import tpu_sc as plsc

_B = 16384
_H = 50
_D = 64
_HP = 56
_DP = 128
_G = 8


@functools.cache
def _build(rows_per_w, nc):
    mesh = plsc.VectorSubcoreMesh(core_axis_name="c", subcore_axis_name="s")
    n2 = rows_per_w // _G // 2

    @functools.partial(
        pl.kernel,
        mesh=mesh,
        out_type=jax.ShapeDtypeStruct((_B, _HP, _DP), jnp.float32),
        compiler_params=pltpu.CompilerParams(use_tc_tiling_on_sc=False),
        scratch_types=[
            pltpu.VMEM((rows_per_w, _H), jnp.int32),
            pltpu.VMEM((_G, _H, _D), jnp.float32),
            pltpu.VMEM((_G, _H, _D), jnp.float32),
            pltpu.SemaphoreType.DMA,
            pltpu.SemaphoreType.DMA,
        ],
    )
    def gather_kernel(idx_hbm, table_hbm, out_hbm, idx_v, buf_a, buf_b, sem_a, sem_b):
        wid = lax.axis_index("s") * nc + lax.axis_index("c")
        row0 = wid * rows_per_w
        pltpu.sync_copy(idx_hbm.at[pl.ds(row0, rows_per_w)], idx_v)

        def gather(local_row, buf, sem):
            return pltpu.make_async_copy(
                table_hbm.at[idx_v.at[local_row]], buf, sem
            )

        def store(buf, row):
            pltpu.sync_copy(buf, out_hbm.at[row, pl.ds(0, _H), pl.ds(0, _D)])

        for j in range(_G):
            gather(j, buf_a.at[j], sem_a).start()

        def body(h, carry):
            g0 = 2 * h
            for j in range(_G):
                gather((g0 + 1) * _G + j, buf_b.at[j], sem_b).start()
            for j in range(_G):
                gather(g0 * _G + j, buf_a.at[j], sem_a).wait()
                store(buf_a.at[j], row0 + g0 * _G + j)
            @pl.when(h + 1 < n2)
            def _():
                for j in range(_G):
                    gather((g0 + 2) * _G + j, buf_a.at[j], sem_a).start()
            for j in range(_G):
                gather((g0 + 1) * _G + j, buf_b.at[j], sem_b).wait()
                store(buf_b.at[j], row0 + (g0 + 1) * _G + j)
            return carry

        lax.fori_loop(0, n2, body, 0)

    return gather_kernel


def kernel(input_, weight):
    info = plsc.get_sparse_core_info()
    nw = info.num_cores * info.num_subcores
    rows_per_w = _B // nw
    wpad = jnp.pad(weight, ((0, 0), (0, _D))).reshape(2 * 1000000, _D)
    idx2 = input_ * 2
    out_p = _build(rows_per_w, info.num_cores)(idx2, wpad)
    return out_p[:, :_H, :_D]
